# baseline (speedup 1.0000x reference)
import functools

import jax
import jax.numpy as jnp
from jax import lax
from jax.experimental import pallas as pl
from jax.experimental.pallas import tpu as pltpu
from jax.experimental.pallas import tpu_sc as plsc

MODEL_DIM = 4096
NUM_EXPERTS = 64
TOP_K = 8
N_TOKENS = 32768

TBLK = 512
NHALF = 2
HTOK = N_TOKENS // NHALF
NSLAB = HTOK // TBLK

NUM_WORKERS = 32
SLABS_PER_TILE = NSLAB // NUM_WORKERS
GROUP = 16


def _matmul_t_body(wt_ref, x_ref, bb_ref, o_ref):
    acc = lax.dot_general(
        wt_ref[...],
        x_ref[...],
        (((1,), (1,)), ((), ())),
        preferred_element_type=jnp.float32,
    )
    o_ref[...] = (acc + bb_ref[...]).reshape(1, NUM_EXPERTS, TBLK)


def _gate_logits_t(x, Wt, bb):
    grid = (NSLAB,)
    return pl.pallas_call(
        _matmul_t_body,
        grid=grid,
        in_specs=[
            pl.BlockSpec((NUM_EXPERTS, MODEL_DIM), lambda i: (0, 0)),
            pl.BlockSpec((TBLK, MODEL_DIM), lambda i: (i, 0)),
            pl.BlockSpec((NUM_EXPERTS, TBLK), lambda i: (0, 0)),
        ],
        out_specs=pl.BlockSpec((1, NUM_EXPERTS, TBLK), lambda i: (i, 0, 0)),
        out_shape=jax.ShapeDtypeStruct((NSLAB, NUM_EXPERTS, TBLK), jnp.float32),
    )(Wt, x, bb)


@functools.partial(
    pl.kernel,
    out_type=jax.ShapeDtypeStruct((NSLAB, NUM_EXPERTS, TBLK), jnp.float32),
    mesh=plsc.VectorSubcoreMesh(
        core_axis_name="c", subcore_axis_name="s", num_cores=2, num_subcores=16
    ),
    scratch_types=[
        pltpu.VMEM((NUM_EXPERTS, TBLK), jnp.float32),
        pltpu.VMEM((NUM_EXPERTS, TBLK), jnp.float32),
    ],
    compiler_params=pltpu.CompilerParams(needs_layout_passes=False),
)
def _sc_routing(lt_hbm, out_hbm, in_v, out_v):
    wid = lax.axis_index("s") * 2 + lax.axis_index("c")

    sort8_net = (
        ((0, 1), (2, 3), (4, 5), (6, 7)),
        ((0, 2), (1, 3), (4, 6), (5, 7)),
        ((1, 2), (5, 6)),
        ((0, 4), (1, 5), (2, 6), (3, 7)),
        ((2, 4), (3, 5)),
        ((1, 2), (3, 4), (5, 6)),
    )
    bitonic8_net = (
        ((0, 4), (1, 5), (2, 6), (3, 7)),
        ((0, 2), (1, 3), (4, 6), (5, 7)),
        ((0, 1), (2, 3), (4, 5), (6, 7)),
    )

    def apply_net(v, net):
        v = list(v)
        for stage in net:
            for i, j in stage:
                hi = jnp.maximum(v[i], v[j])
                lo = jnp.minimum(v[i], v[j])
                v[i], v[j] = hi, lo
        return v

    def slab_body(si, carry):
        sl = wid * SLABS_PER_TILE + si
        pltpu.sync_copy(lt_hbm.at[sl], in_v)

        def group_body(g, gcarry):
            t0 = g * GROUP

            def load_batch(b):
                return [
                    in_v[8 * b + e, pl.ds(t0, GROUP)] for e in range(8)
                ]

            m = apply_net(load_batch(0), sort8_net)
            for b in range(1, 7):
                s = apply_net(load_batch(b), sort8_net)
                c = [jnp.maximum(m[i], s[7 - i]) for i in range(8)]
                m = apply_net(c, bitonic8_net)
            s = apply_net(load_batch(7), sort8_net)
            mx = jnp.maximum(m[0], s[0])
            c = [jnp.maximum(m[i], s[7 - i]) for i in range(8)]
            thr = jnp.minimum(
                jnp.minimum(jnp.minimum(c[0], c[1]), jnp.minimum(c[2], c[3])),
                jnp.minimum(jnp.minimum(c[4], c[5]), jnp.minimum(c[6], c[7])),
            )
            ssum = (
                (jnp.exp(c[0] - mx) + jnp.exp(c[1] - mx))
                + (jnp.exp(c[2] - mx) + jnp.exp(c[3] - mx))
            ) + (
                (jnp.exp(c[4] - mx) + jnp.exp(c[5] - mx))
                + (jnp.exp(c[6] - mx) + jnp.exp(c[7] - mx))
            )
            inv = 1.0 / ssum
            for e in range(NUM_EXPERTS):
                v = in_v[e, pl.ds(t0, GROUP)]
                w = jnp.where(v >= thr, jnp.exp(v - mx) * inv, 0.0)
                out_v[e, pl.ds(t0, GROUP)] = w
            return gcarry

        lax.fori_loop(0, TBLK // GROUP, group_body, 0)
        pltpu.sync_copy(out_v, out_hbm.at[sl])
        return carry

    lax.fori_loop(0, SLABS_PER_TILE, slab_body, 0)


def kernel(x, W, b):
    Wt = W.T
    bb = jnp.broadcast_to(b.reshape(NUM_EXPERTS, 1), (NUM_EXPERTS, TBLK))
    outs = []
    for h in range(NHALF):
        xh = lax.slice_in_dim(x, h * HTOK, (h + 1) * HTOK, axis=0)
        lt = _gate_logits_t(xh, Wt, bb)
        ot = _sc_routing(lt)
        outs.append(ot.transpose(0, 2, 1).reshape(HTOK, NUM_EXPERTS))
    return jnp.concatenate(outs, axis=0)


# speedup vs baseline: 2.5105x; 2.5105x over previous
import functools

import jax
import jax.numpy as jnp
from jax import lax
from jax.experimental import pallas as pl
from jax.experimental.pallas import tpu as pltpu
from jax.experimental.pallas import tpu_sc as plsc

MODEL_DIM = 4096
NUM_EXPERTS = 64
TOP_K = 8
N_TOKENS = 32768

TBLK = 512
NHALF = 2
HTOK = N_TOKENS // NHALF
NSLAB = HTOK // TBLK

NUM_WORKERS = 32
SLABS_PER_TILE = NSLAB // NUM_WORKERS
GROUP = 16


def _matmul_t_body(wt_ref, x_ref, bb_ref, o_ref):
    acc = lax.dot_general(
        wt_ref[...],
        x_ref[...],
        (((1,), (1,)), ((), ())),
        preferred_element_type=jnp.float32,
    )
    o_ref[...] = (acc + bb_ref[...]).reshape(1, NUM_EXPERTS, TBLK)


def _gate_logits_t(x, Wt, bb, h):
    grid = (NSLAB,)
    blk0 = h * NSLAB
    return pl.pallas_call(
        _matmul_t_body,
        grid=grid,
        in_specs=[
            pl.BlockSpec((NUM_EXPERTS, MODEL_DIM), lambda i: (0, 0)),
            pl.BlockSpec((TBLK, MODEL_DIM), lambda i: (blk0 + i, 0)),
            pl.BlockSpec((NUM_EXPERTS, TBLK), lambda i: (0, 0)),
        ],
        out_specs=pl.BlockSpec((1, NUM_EXPERTS, TBLK), lambda i: (i, 0, 0)),
        out_shape=jax.ShapeDtypeStruct((NSLAB, NUM_EXPERTS, TBLK), jnp.float32),
    )(Wt, x, bb)


@functools.partial(
    pl.kernel,
    out_type=jax.ShapeDtypeStruct((NSLAB, NUM_EXPERTS, TBLK), jnp.float32),
    mesh=plsc.VectorSubcoreMesh(
        core_axis_name="c", subcore_axis_name="s", num_cores=2, num_subcores=16
    ),
    scratch_types=[
        pltpu.VMEM((NUM_EXPERTS, TBLK), jnp.float32),
        pltpu.VMEM((NUM_EXPERTS, TBLK), jnp.float32),
    ],
    compiler_params=pltpu.CompilerParams(needs_layout_passes=False),
)
def _sc_routing(lt_hbm, out_hbm, in_v, out_v):
    wid = lax.axis_index("s") * 2 + lax.axis_index("c")

    sort8_net = (
        ((0, 1), (2, 3), (4, 5), (6, 7)),
        ((0, 2), (1, 3), (4, 6), (5, 7)),
        ((1, 2), (5, 6)),
        ((0, 4), (1, 5), (2, 6), (3, 7)),
        ((2, 4), (3, 5)),
        ((1, 2), (3, 4), (5, 6)),
    )
    bitonic8_net = (
        ((0, 4), (1, 5), (2, 6), (3, 7)),
        ((0, 2), (1, 3), (4, 6), (5, 7)),
        ((0, 1), (2, 3), (4, 5), (6, 7)),
    )

    def apply_net(v, net):
        v = list(v)
        for stage in net:
            for i, j in stage:
                hi = jnp.maximum(v[i], v[j])
                lo = jnp.minimum(v[i], v[j])
                v[i], v[j] = hi, lo
        return v

    def slab_body(si, carry):
        sl = wid * SLABS_PER_TILE + si
        pltpu.sync_copy(lt_hbm.at[sl], in_v)

        def group_body(g, gcarry):
            t0 = g * GROUP

            def load_batch(b):
                return [
                    in_v[8 * b + e, pl.ds(t0, GROUP)] for e in range(8)
                ]

            m = apply_net(load_batch(0), sort8_net)
            for b in range(1, 7):
                s = apply_net(load_batch(b), sort8_net)
                c = [jnp.maximum(m[i], s[7 - i]) for i in range(8)]
                m = apply_net(c, bitonic8_net)
            s = apply_net(load_batch(7), sort8_net)
            mx = jnp.maximum(m[0], s[0])
            c = [jnp.maximum(m[i], s[7 - i]) for i in range(8)]
            thr = jnp.minimum(
                jnp.minimum(jnp.minimum(c[0], c[1]), jnp.minimum(c[2], c[3])),
                jnp.minimum(jnp.minimum(c[4], c[5]), jnp.minimum(c[6], c[7])),
            )
            ssum = (
                (jnp.exp(c[0] - mx) + jnp.exp(c[1] - mx))
                + (jnp.exp(c[2] - mx) + jnp.exp(c[3] - mx))
            ) + (
                (jnp.exp(c[4] - mx) + jnp.exp(c[5] - mx))
                + (jnp.exp(c[6] - mx) + jnp.exp(c[7] - mx))
            )
            inv = 1.0 / ssum
            for e in range(NUM_EXPERTS):
                v = in_v[e, pl.ds(t0, GROUP)]
                w = jnp.where(v >= thr, jnp.exp(v - mx) * inv, 0.0)
                out_v[e, pl.ds(t0, GROUP)] = w
            return gcarry

        lax.fori_loop(0, TBLK // GROUP, group_body, 0)
        pltpu.sync_copy(out_v, out_hbm.at[sl])
        return carry

    lax.fori_loop(0, SLABS_PER_TILE, slab_body, 0)


def kernel(x, W, b):
    Wt = W.T
    bb = jnp.broadcast_to(b.reshape(NUM_EXPERTS, 1), (NUM_EXPERTS, TBLK))
    outs = []
    for h in range(NHALF):
        lt = _gate_logits_t(x, Wt, bb, h)
        ot = _sc_routing(lt)
        outs.append(ot.transpose(0, 2, 1).reshape(HTOK, NUM_EXPERTS))
    return jnp.concatenate(outs, axis=0)


# speedup vs baseline: 2.6737x; 1.0650x over previous
import functools

import jax
import jax.numpy as jnp
from jax import lax
from jax.experimental import pallas as pl
from jax.experimental.pallas import tpu as pltpu
from jax.experimental.pallas import tpu_sc as plsc

MODEL_DIM = 4096
NUM_EXPERTS = 64
TOP_K = 8
N_TOKENS = 32768

TBLK = 512
NHALF = 4
HTOK = N_TOKENS // NHALF
NSLAB = HTOK // TBLK

NUM_WORKERS = 32
SLABS_PER_TILE = NSLAB // NUM_WORKERS
GROUP = 16


def _matmul_t_body(wt_ref, x_ref, bb_ref, o_ref):
    acc = lax.dot_general(
        wt_ref[...],
        x_ref[...],
        (((1,), (1,)), ((), ())),
        preferred_element_type=jnp.float32,
    )
    o_ref[...] = (acc + bb_ref[...]).reshape(1, NUM_EXPERTS, TBLK)


def _gate_logits_t(x, Wt, bb, h):
    grid = (NSLAB,)
    blk0 = h * NSLAB
    return pl.pallas_call(
        _matmul_t_body,
        grid=grid,
        in_specs=[
            pl.BlockSpec((NUM_EXPERTS, MODEL_DIM), lambda i: (0, 0)),
            pl.BlockSpec((TBLK, MODEL_DIM), lambda i: (blk0 + i, 0)),
            pl.BlockSpec((NUM_EXPERTS, TBLK), lambda i: (0, 0)),
        ],
        out_specs=pl.BlockSpec((1, NUM_EXPERTS, TBLK), lambda i: (i, 0, 0)),
        out_shape=jax.ShapeDtypeStruct((NSLAB, NUM_EXPERTS, TBLK), jnp.float32),
    )(Wt, x, bb)


@functools.partial(
    pl.kernel,
    out_type=jax.ShapeDtypeStruct((NSLAB, NUM_EXPERTS, TBLK), jnp.float32),
    mesh=plsc.VectorSubcoreMesh(
        core_axis_name="c", subcore_axis_name="s", num_cores=2, num_subcores=16
    ),
    scratch_types=[
        pltpu.VMEM((NUM_EXPERTS, TBLK), jnp.float32),
        pltpu.VMEM((NUM_EXPERTS, TBLK), jnp.float32),
    ],
    compiler_params=pltpu.CompilerParams(needs_layout_passes=False),
)
def _sc_routing(lt_hbm, out_hbm, in_v, out_v):
    wid = lax.axis_index("s") * 2 + lax.axis_index("c")

    sort8_net = (
        ((0, 1), (2, 3), (4, 5), (6, 7)),
        ((0, 2), (1, 3), (4, 6), (5, 7)),
        ((1, 2), (5, 6)),
        ((0, 4), (1, 5), (2, 6), (3, 7)),
        ((2, 4), (3, 5)),
        ((1, 2), (3, 4), (5, 6)),
    )
    bitonic8_net = (
        ((0, 4), (1, 5), (2, 6), (3, 7)),
        ((0, 2), (1, 3), (4, 6), (5, 7)),
        ((0, 1), (2, 3), (4, 5), (6, 7)),
    )

    def apply_net(v, net):
        v = list(v)
        for stage in net:
            for i, j in stage:
                hi = jnp.maximum(v[i], v[j])
                lo = jnp.minimum(v[i], v[j])
                v[i], v[j] = hi, lo
        return v

    def slab_body(si, carry):
        sl = wid * SLABS_PER_TILE + si
        pltpu.sync_copy(lt_hbm.at[sl], in_v)

        def group_body(g, gcarry):
            t0 = g * GROUP

            def load_batch(b):
                return [
                    in_v[8 * b + e, pl.ds(t0, GROUP)] for e in range(8)
                ]

            m = apply_net(load_batch(0), sort8_net)
            for b in range(1, 7):
                s = apply_net(load_batch(b), sort8_net)
                c = [jnp.maximum(m[i], s[7 - i]) for i in range(8)]
                m = apply_net(c, bitonic8_net)
            s = apply_net(load_batch(7), sort8_net)
            mx = jnp.maximum(m[0], s[0])
            c = [jnp.maximum(m[i], s[7 - i]) for i in range(8)]
            thr = jnp.minimum(
                jnp.minimum(jnp.minimum(c[0], c[1]), jnp.minimum(c[2], c[3])),
                jnp.minimum(jnp.minimum(c[4], c[5]), jnp.minimum(c[6], c[7])),
            )
            ssum = (
                (jnp.exp(c[0] - mx) + jnp.exp(c[1] - mx))
                + (jnp.exp(c[2] - mx) + jnp.exp(c[3] - mx))
            ) + (
                (jnp.exp(c[4] - mx) + jnp.exp(c[5] - mx))
                + (jnp.exp(c[6] - mx) + jnp.exp(c[7] - mx))
            )
            inv = 1.0 / ssum
            for e in range(NUM_EXPERTS):
                v = in_v[e, pl.ds(t0, GROUP)]
                w = jnp.where(v >= thr, jnp.exp(v - mx) * inv, 0.0)
                out_v[e, pl.ds(t0, GROUP)] = w
            return gcarry

        lax.fori_loop(0, TBLK // GROUP, group_body, 0)
        pltpu.sync_copy(out_v, out_hbm.at[sl])
        return carry

    lax.fori_loop(0, SLABS_PER_TILE, slab_body, 0)


def kernel(x, W, b):
    Wt = W.T
    bb = jnp.broadcast_to(b.reshape(NUM_EXPERTS, 1), (NUM_EXPERTS, TBLK))
    outs = []
    for h in range(NHALF):
        lt = _gate_logits_t(x, Wt, bb, h)
        ot = _sc_routing(lt)
        outs.append(ot.transpose(0, 2, 1).reshape(HTOK, NUM_EXPERTS))
    return jnp.concatenate(outs, axis=0)
